# baseline (speedup 1.0000x reference)
import jax
import jax.numpy as jnp
from jax import lax
from jax.experimental import pallas as pl
from jax.experimental.pallas import tpu as pltpu
from jax.experimental.pallas import tpu_sc as plsc

_LANES = 16
_CPAD = 1216
_NUM_SAMPLE = 50


def _sc_mask_body(label_hbm, g_hbm, out_hbm, lbl_v, g_v, mask_v):
    cid = lax.axis_index("c")
    sid = lax.axis_index("s")
    n_lbl = label_hbm.shape[0]

    @pl.when(jnp.logical_and(cid == 0, sid == 0))
    def _():
        pltpu.sync_copy(label_hbm, lbl_v)
        pltpu.sync_copy(g_hbm, g_v)

        zeros16 = jnp.zeros((_LANES,), jnp.float32)
        ones16 = jnp.ones((_LANES,), jnp.float32)
        iota16 = lax.iota(jnp.int32, _LANES)

        def zero_chunk(i, c):
            mask_v[pl.ds(i * _LANES, _LANES)] = zeros16
            return c

        lax.fori_loop(0, _CPAD // _LANES, zero_chunk, 0)

        def scatter_chunk(i, c):
            idx = lbl_v[pl.ds(i * _LANES, _LANES)]
            plsc.store_scatter(mask_v, [idx], ones16)
            return c

        lax.fori_loop(0, n_lbl // _LANES, scatter_chunk, 0)

        def count_chunk(i, acc):
            return acc + mask_v[pl.ds(i * _LANES, _LANES)]

        acc = lax.fori_loop(0, _CPAD // _LANES, count_chunk, zeros16)
        n_app = jnp.sum(acc)
        k_need = jnp.float32(_NUM_SAMPLE) - n_app

        @pl.when(n_app < jnp.float32(_NUM_SAMPLE))
        def _sample():
            neg_inf = jnp.full((_LANES,), -jnp.inf, jnp.float32)

            def extract_one(i, c):
                @pl.when(jnp.float32(i) < k_need)
                def _():
                    def scan_chunk(j, carry):
                        bv, bi = carry
                        gv = g_v[pl.ds(j * _LANES, _LANES)]
                        mv = mask_v[pl.ds(j * _LANES, _LANES)]
                        ge = jnp.where(mv > 0.0, neg_inf, gv)
                        upd = ge > bv
                        bv = jnp.where(upd, ge, bv)
                        bi = jnp.where(upd, iota16 + j * _LANES, bi)
                        return bv, bi

                    bv, bi = lax.fori_loop(
                        0, _CPAD // _LANES, scan_chunk,
                        (neg_inf, jnp.zeros((_LANES,), jnp.int32)))
                    mx = jnp.max(bv)
                    sel = jnp.sum(jnp.where(bv == mx, bi, 0))
                    idxv = jnp.full((_LANES,), sel, jnp.int32)
                    plsc.store_scatter(mask_v, [idxv], ones16,
                                       mask=iota16 == 0)

                return c

            lax.fori_loop(0, _NUM_SAMPLE - 1, extract_one, 0)

        pltpu.sync_copy(mask_v, out_hbm)


def _sc_mask(label, g_padded):
    mesh = plsc.VectorSubcoreMesh(core_axis_name="c", subcore_axis_name="s")
    fn = pl.kernel(
        _sc_mask_body,
        out_type=jax.ShapeDtypeStruct((_CPAD,), jnp.float32),
        scratch_types=[
            pltpu.VMEM((label.shape[0],), jnp.int32),
            pltpu.VMEM((_CPAD,), jnp.float32),
            pltpu.VMEM((_CPAD,), jnp.float32),
        ],
        mesh=mesh,
        compiler_params=pltpu.CompilerParams(needs_layout_passes=False),
    )
    return fn(label, g_padded)


def _tc_colsum_body(score_ref, label_ref, cols_ref, out_ref):
    i = pl.program_id(0)
    x = jnp.clip(score_ref[...], -4.0, 10.0)
    log2e = jnp.float32(1.4426950408889634)
    e = lax.exp2(x * (-log2e))
    l1p = jnp.log(1.0 - lax.exp2(e * (-log2e)))
    val = jnp.where(cols_ref[...] == label_ref[...], -e, l1p)
    ones_row = jnp.ones((1, val.shape[0]), jnp.float32)
    colpart = jax.lax.dot_general(
        ones_row, val, (((1,), (0,)), ((), ())),
        preferred_element_type=jnp.float32)

    @pl.when(i == 0)
    def _():
        out_ref[...] = colpart

    @pl.when(i > 0)
    def _():
        out_ref[...] += colpart


def _tc_colsum(cls_score, label2d, cols2d, block_rows=2048):
    b, c = cls_score.shape
    grid = b // block_rows
    return pl.pallas_call(
        _tc_colsum_body,
        grid=(grid,),
        in_specs=[
            pl.BlockSpec((block_rows, c), lambda i: (i, 0)),
            pl.BlockSpec((block_rows, 1), lambda i: (i, 0)),
            pl.BlockSpec((1, c), lambda i: (0, 0)),
        ],
        out_specs=pl.BlockSpec((1, c), lambda i: (0, 0)),
        out_shape=jax.ShapeDtypeStruct((1, c), jnp.float32),
        compiler_params=pltpu.CompilerParams(
            dimension_semantics=("arbitrary",),
            allow_input_fusion=[True, True, True]),
    )(cls_score, label2d, cols2d)


def _tc_combine_body(colsum_ref, mask_ref, nrows_ref, out_ref):
    s = jnp.sum(colsum_ref[...] * mask_ref[...])
    out_ref[0, 0] = s * (-1.0 / jnp.float32(nrows_ref[0]))


def _tc_combine(colsum, mask2d, nrows):
    c = colsum.shape[1]
    return pl.pallas_call(
        _tc_combine_body,
        grid=(1,),
        in_specs=[
            pl.BlockSpec((1, c), lambda i: (0, 0)),
            pl.BlockSpec((1, c), lambda i: (0, 0)),
            pl.BlockSpec(memory_space=pltpu.SMEM),
        ],
        out_specs=pl.BlockSpec((1, 1), lambda i: (0, 0),
                               memory_space=pltpu.SMEM),
        out_shape=jax.ShapeDtypeStruct((1, 1), jnp.float32),
    )(colsum, mask2d, nrows)


def kernel(cls_score, label, freq_weight):
    b, c = cls_score.shape
    gum = jax.random.gumbel(jax.random.key(1), (c + 1,), jnp.float32)
    g = jnp.concatenate([
        gum[:c] + jnp.log(freq_weight),
        jnp.full((_CPAD - c,), -jnp.inf, jnp.float32),
    ])
    mask_full = _sc_mask(label, g)
    mask2d = mask_full[:c].reshape(1, c)
    cols2d = jnp.arange(c, dtype=jnp.int32).reshape(1, c)
    label2d = label.reshape(b, 1)
    nchunks = 4
    rows = b // nchunks
    colsum = None
    for j in range(nchunks):
        part = _tc_colsum(
            lax.slice(cls_score, (j * rows, 0), ((j + 1) * rows, c)),
            lax.slice(label2d, (j * rows, 0), ((j + 1) * rows, 1)),
            cols2d)
        colsum = part if colsum is None else colsum + part
    out = _tc_combine(colsum, mask2d, jnp.array([b], jnp.int32))
    return out[0, 0]


# speedup vs baseline: 1.7311x; 1.7311x over previous
import jax
import jax.numpy as jnp
from jax import lax
from jax.experimental import pallas as pl
from jax.experimental.pallas import tpu as pltpu
from jax.experimental.pallas import tpu_sc as plsc

_LANES = 16
_CPAD = 1216
_NUM_SAMPLE = 50


def _sc_mask_body(label_hbm, g_hbm, out_hbm, lbl_v, g_v, mask_v):
    cid = lax.axis_index("c")
    sid = lax.axis_index("s")
    n_lbl = label_hbm.shape[0]

    @pl.when(jnp.logical_and(cid == 0, sid == 0))
    def _():
        pltpu.sync_copy(label_hbm, lbl_v)
        pltpu.sync_copy(g_hbm, g_v)

        zeros16 = jnp.zeros((_LANES,), jnp.float32)
        ones16 = jnp.ones((_LANES,), jnp.float32)
        iota16 = lax.iota(jnp.int32, _LANES)

        def zero_chunk(i, c):
            mask_v[pl.ds(i * _LANES, _LANES)] = zeros16
            return c

        lax.fori_loop(0, _CPAD // _LANES, zero_chunk, 0)

        def scatter_chunk(i, c):
            idx = lbl_v[pl.ds(i * _LANES, _LANES)]
            plsc.store_scatter(mask_v, [idx], ones16)
            return c

        lax.fori_loop(0, n_lbl // _LANES, scatter_chunk, 0)

        def count_chunk(i, acc):
            return acc + mask_v[pl.ds(i * _LANES, _LANES)]

        acc = lax.fori_loop(0, _CPAD // _LANES, count_chunk, zeros16)
        n_app = jnp.sum(acc)
        k_need = jnp.float32(_NUM_SAMPLE) - n_app

        @pl.when(n_app < jnp.float32(_NUM_SAMPLE))
        def _sample():
            neg_inf = jnp.full((_LANES,), -jnp.inf, jnp.float32)

            def extract_one(i, c):
                @pl.when(jnp.float32(i) < k_need)
                def _():
                    def scan_chunk(j, carry):
                        bv, bi = carry
                        gv = g_v[pl.ds(j * _LANES, _LANES)]
                        mv = mask_v[pl.ds(j * _LANES, _LANES)]
                        ge = jnp.where(mv > 0.0, neg_inf, gv)
                        upd = ge > bv
                        bv = jnp.where(upd, ge, bv)
                        bi = jnp.where(upd, iota16 + j * _LANES, bi)
                        return bv, bi

                    bv, bi = lax.fori_loop(
                        0, _CPAD // _LANES, scan_chunk,
                        (neg_inf, jnp.zeros((_LANES,), jnp.int32)))
                    mx = jnp.max(bv)
                    sel = jnp.sum(jnp.where(bv == mx, bi, 0))
                    idxv = jnp.full((_LANES,), sel, jnp.int32)
                    plsc.store_scatter(mask_v, [idxv], ones16,
                                       mask=iota16 == 0)

                return c

            lax.fori_loop(0, _NUM_SAMPLE - 1, extract_one, 0)

        pltpu.sync_copy(mask_v, out_hbm)


def _sc_mask(label, g_padded):
    mesh = plsc.VectorSubcoreMesh(core_axis_name="c", subcore_axis_name="s")
    fn = pl.kernel(
        _sc_mask_body,
        out_type=jax.ShapeDtypeStruct((_CPAD,), jnp.float32),
        scratch_types=[
            pltpu.VMEM((label.shape[0],), jnp.int32),
            pltpu.VMEM((_CPAD,), jnp.float32),
            pltpu.VMEM((_CPAD,), jnp.float32),
        ],
        mesh=mesh,
        compiler_params=pltpu.CompilerParams(needs_layout_passes=False),
    )
    return fn(label, g_padded)


def _tc_colsum_body(score_ref, label_ref, cols_ref, out_ref):
    i = pl.program_id(0)
    x = jnp.clip(score_ref[...], -4.0, 10.0)
    log2e = jnp.float32(1.4426950408889634)
    e = lax.exp2(x * (-log2e))
    l1p = jnp.log(1.0 - lax.exp2(e * (-log2e)))
    val = jnp.where(cols_ref[...] == label_ref[...], -e, l1p)
    ones_row = jnp.ones((1, val.shape[0]), jnp.float32)
    colpart = jax.lax.dot_general(
        ones_row, val, (((1,), (0,)), ((), ())),
        preferred_element_type=jnp.float32)

    @pl.when(i == 0)
    def _():
        out_ref[...] = colpart

    @pl.when(i > 0)
    def _():
        out_ref[...] += colpart


def _tc_colsum(cls_score, label2d, cols2d, block_rows=2048):
    b, c = cls_score.shape
    grid = b // block_rows
    return pl.pallas_call(
        _tc_colsum_body,
        grid=(grid,),
        in_specs=[
            pl.BlockSpec((block_rows, c), lambda i: (i, 0)),
            pl.BlockSpec((block_rows, 1), lambda i: (i, 0)),
            pl.BlockSpec((1, c), lambda i: (0, 0)),
        ],
        out_specs=pl.BlockSpec((1, c), lambda i: (0, 0)),
        out_shape=jax.ShapeDtypeStruct((1, c), jnp.float32),
        compiler_params=pltpu.CompilerParams(
            dimension_semantics=("arbitrary",),
            allow_input_fusion=[True, True, True]),
    )(cls_score, label2d, cols2d)


def _tc_combine_body(colsum_ref, mask_ref, nrows_ref, out_ref):
    s = jnp.sum(colsum_ref[...] * mask_ref[...])
    out_ref[0, 0] = s * (-1.0 / jnp.float32(nrows_ref[0]))


def _tc_combine(colsum, mask2d, nrows):
    c = colsum.shape[1]
    return pl.pallas_call(
        _tc_combine_body,
        grid=(1,),
        in_specs=[
            pl.BlockSpec((1, c), lambda i: (0, 0)),
            pl.BlockSpec((1, c), lambda i: (0, 0)),
            pl.BlockSpec(memory_space=pltpu.SMEM),
        ],
        out_specs=pl.BlockSpec((1, 1), lambda i: (0, 0),
                               memory_space=pltpu.SMEM),
        out_shape=jax.ShapeDtypeStruct((1, 1), jnp.float32),
    )(colsum, mask2d, nrows)


def kernel(cls_score, label, freq_weight):
    b, c = cls_score.shape
    gum = jax.random.gumbel(jax.random.key(1), (c + 1,), jnp.float32)
    g = jnp.concatenate([
        gum[:c] + jnp.log(freq_weight),
        jnp.full((_CPAD - c,), -jnp.inf, jnp.float32),
    ])
    mask_full = _sc_mask(label, g)
    mask2d = mask_full[:c].reshape(1, c)
    cols2d = jnp.arange(c, dtype=jnp.int32).reshape(1, c)
    colsum = _tc_colsum(cls_score, label.reshape(b, 1), cols2d)
    out = _tc_combine(colsum, mask2d, jnp.array([b], jnp.int32))
    return out[0, 0]
